# baseline (speedup 1.0000x reference)
import jax
import jax.numpy as jnp
from jax.experimental import pallas as pl
from jax.experimental.pallas import tpu as pltpu

_INV_SQRT2 = 0.7071067811865476


def _routing_body(c_ref, wt_ref, b_ref, idx_ref, wts_ref):
    c_mean = jnp.mean(c_ref[...], axis=0, keepdims=True)
    logits = jnp.dot(
        c_mean.astype(jnp.bfloat16),
        wt_ref[...].astype(jnp.bfloat16),
        preferred_element_type=jnp.float32,
    ) + b_ref[...]
    lane = jax.lax.broadcasted_iota(jnp.int32, logits.shape, 1)
    m1 = jnp.max(logits)
    i1 = jnp.min(jnp.where(logits == m1, lane, logits.shape[1]))
    masked = jnp.where(lane == i1, -jnp.inf, logits)
    m2 = jnp.max(masked)
    i2 = jnp.min(jnp.where(masked == m2, lane, logits.shape[1]))
    e = jnp.exp(m2 - m1)
    w0 = 1.0 / (1.0 + e)
    w1 = e / (1.0 + e)
    pos = jax.lax.broadcasted_iota(jnp.int32, (1, 2), 1)
    idx_ref[...] = jnp.where(pos == 0, i1, i2)
    wts_ref[...] = jnp.where(pos == 0, w0, w1)


def _compact_body(s_ref, w1_ref, w2_ref, b1_ref, b2_ref,
                  w1c_ref, w2c_ref, b1c_ref, b2c_ref):
    del s_ref
    w1c_ref[...] = w1_ref[...].astype(jnp.bfloat16)
    w2c_ref[...] = w2_ref[...].astype(jnp.bfloat16)
    b1c_ref[...] = b1_ref[...]
    b2c_ref[...] = b2_ref[...]


def _moe_body(x_ref, w1a_ref, w1b_ref, w2a_ref, w2b_ref,
              b1a_ref, b1b_ref, b2a_ref, b2b_ref, wts_ref, out_ref):
    xv = x_ref[...].astype(jnp.bfloat16)

    def expert(w1_ref, b1_ref, w2_ref, b2_ref):
        h = jnp.dot(xv, w1_ref[0], preferred_element_type=jnp.float32)
        h = h + b1_ref[0]
        h = 0.5 * h * (1.0 + jax.lax.erf(h * _INV_SQRT2))
        return jnp.dot(h.astype(jnp.bfloat16), w2_ref[0],
                       preferred_element_type=jnp.float32) + b2_ref[0]

    w0 = wts_ref[0]
    w1 = wts_ref[1]
    out_ref[...] = (expert(w1a_ref, b1a_ref, w2a_ref, b2a_ref) * w0
                    + expert(w1b_ref, b1b_ref, w2b_ref, b2b_ref) * w1)


@jax.jit
def kernel(x, c_states, router_W, router_b, W1, b1, W2, b2):
    B, T, D = x.shape
    E, _, H = W1.shape
    M = B * T
    BM = 512
    ND = 4

    idx2, wts2 = pl.pallas_call(
        _routing_body,
        out_shape=(
            jax.ShapeDtypeStruct((1, 2), jnp.int32),
            jax.ShapeDtypeStruct((1, 2), jnp.float32),
        ),
    )(c_states, router_W.T, router_b.reshape(1, E))
    idx = idx2.reshape(2)
    wts = wts2.reshape(2)

    b1r = b1.reshape(E, 1, H)
    b2r = b2.reshape(E, 1, D)

    compact_spec = pltpu.PrefetchScalarGridSpec(
        num_scalar_prefetch=1,
        grid=(2, ND),
        in_specs=[
            pl.BlockSpec((1, D // ND, H), lambda e, d, s: (s[e], d, 0)),
            pl.BlockSpec((1, H // ND, D), lambda e, d, s: (s[e], d, 0)),
            pl.BlockSpec((1, 1, H), lambda e, d, s: (s[e], 0, 0)),
            pl.BlockSpec((1, 1, D), lambda e, d, s: (s[e], 0, 0)),
        ],
        out_specs=[
            pl.BlockSpec((1, D // ND, H), lambda e, d, s: (e, d, 0)),
            pl.BlockSpec((1, H // ND, D), lambda e, d, s: (e, d, 0)),
            pl.BlockSpec((1, 1, H), lambda e, d, s: (e, 0, 0)),
            pl.BlockSpec((1, 1, D), lambda e, d, s: (e, 0, 0)),
        ],
    )
    W1c, W2c, b1c, b2c = pl.pallas_call(
        _compact_body,
        grid_spec=compact_spec,
        out_shape=(
            jax.ShapeDtypeStruct((2, D, H), jnp.bfloat16),
            jax.ShapeDtypeStruct((2, H, D), jnp.bfloat16),
            jax.ShapeDtypeStruct((2, 1, H), jnp.float32),
            jax.ShapeDtypeStruct((2, 1, D), jnp.float32),
        ),
    )(idx, W1, W2, b1r, b2r)

    x2 = x.reshape(M, D)
    out = pl.pallas_call(
        _moe_body,
        grid=(M // BM,),
        in_specs=[
            pl.BlockSpec((BM, D), lambda i: (i, 0)),
            pl.BlockSpec((1, D, H), lambda i: (0, 0, 0)),
            pl.BlockSpec((1, D, H), lambda i: (1, 0, 0)),
            pl.BlockSpec((1, H, D), lambda i: (0, 0, 0)),
            pl.BlockSpec((1, H, D), lambda i: (1, 0, 0)),
            pl.BlockSpec((1, 1, H), lambda i: (0, 0, 0)),
            pl.BlockSpec((1, 1, H), lambda i: (1, 0, 0)),
            pl.BlockSpec((1, 1, D), lambda i: (0, 0, 0)),
            pl.BlockSpec((1, 1, D), lambda i: (1, 0, 0)),
            pl.BlockSpec(memory_space=pltpu.SMEM),
        ],
        out_specs=pl.BlockSpec((BM, D), lambda i: (i, 0)),
        out_shape=jax.ShapeDtypeStruct((M, D), jnp.float32),
        compiler_params=pltpu.CompilerParams(
            dimension_semantics=("arbitrary",),
        ),
    )(x2, W1c, W1c, W2c, W2c, b1c, b1c, b2c, b2c, wts)
    return out.reshape(B, T, D)


# speedup vs baseline: 1.0612x; 1.0612x over previous
import jax
import jax.numpy as jnp
from jax.experimental import pallas as pl
from jax.experimental.pallas import tpu as pltpu

_INV_SQRT2 = 0.7071067811865476


def _routing_body(c_ref, wt_ref, b_ref, idx_ref, wts_ref):
    c_mean = jnp.mean(c_ref[...], axis=0, keepdims=True)
    logits = jnp.dot(
        c_mean.astype(jnp.bfloat16),
        wt_ref[...].astype(jnp.bfloat16),
        preferred_element_type=jnp.float32,
    ) + b_ref[...]
    lane = jax.lax.broadcasted_iota(jnp.int32, logits.shape, 1)
    m1 = jnp.max(logits)
    i1 = jnp.min(jnp.where(logits == m1, lane, logits.shape[1]))
    masked = jnp.where(lane == i1, -jnp.inf, logits)
    m2 = jnp.max(masked)
    i2 = jnp.min(jnp.where(masked == m2, lane, logits.shape[1]))
    e = jnp.exp(m2 - m1)
    w0 = 1.0 / (1.0 + e)
    w1 = e / (1.0 + e)
    pos = jax.lax.broadcasted_iota(jnp.int32, (1, 2), 1)
    idx_ref[...] = jnp.where(pos == 0, i1, i2)
    wts_ref[...] = jnp.where(pos == 0, w0, w1)


def _make_moe_body(D, H, NCH):
    CH = H // NCH

    def _moe_body(idx_ref, wts_ref, x_ref, w1_hbm, w2_hbm, b1_hbm, b2_hbm,
                  out_ref, w1a, w1b, w2a, w2b, b1s, b2s, stage, sems, bsem):
        i = pl.program_id(0)

        @pl.when(i == 0)
        def _prologue():
            e0 = idx_ref[0]
            e1 = idx_ref[1]
            bias_copies = [
                pltpu.make_async_copy(b1_hbm.at[e0], b1s.at[0], bsem),
                pltpu.make_async_copy(b1_hbm.at[e1], b1s.at[1], bsem),
                pltpu.make_async_copy(b2_hbm.at[e0], b2s.at[0], bsem),
                pltpu.make_async_copy(b2_hbm.at[e1], b2s.at[1], bsem),
            ]
            for c in bias_copies:
                c.start()
            chunks = []
            for e, w1d, w2d in ((e0, w1a, w2a), (e1, w1b, w2b)):
                for c in range(NCH):
                    sl = pl.ds(c * CH, CH)
                    chunks.append((w1_hbm.at[e, :, sl], w1d,
                                   (slice(None), sl)))
                for c in range(NCH):
                    sl = pl.ds(c * CH, CH)
                    chunks.append((w2_hbm.at[e, sl, :], w2d,
                                   (sl, slice(None))))
            copies = []
            for k, (src, _, _) in enumerate(chunks):
                copies.append(
                    pltpu.make_async_copy(src, stage.at[k % 2], sems.at[k % 2]))
            for k, (_, dst, dsl) in enumerate(chunks):
                copies[k].start()
                if k > 0:
                    copies[k - 1].wait()
                    _, pdst, pdsl = chunks[k - 1]
                    pdst[pdsl] = stage[(k - 1) % 2].astype(jnp.bfloat16)
            copies[-1].wait()
            _, ldst, ldsl = chunks[-1]
            ldst[ldsl] = stage[(len(chunks) - 1) % 2].astype(jnp.bfloat16)
            for c in bias_copies:
                c.wait()

        xv = x_ref[...].astype(jnp.bfloat16)

        def expert(w1_s, b1_i, w2_s, b2_i):
            h = jnp.dot(xv, w1_s[...], preferred_element_type=jnp.float32)
            h = h + b1s[b1_i]
            h = 0.5 * h * (1.0 + jax.lax.erf(h * _INV_SQRT2))
            return jnp.dot(h.astype(jnp.bfloat16), w2_s[...],
                           preferred_element_type=jnp.float32) + b2s[b2_i]

        w0 = wts_ref[0]
        w1 = wts_ref[1]
        out_ref[...] = (expert(w1a, 0, w2a, 0) * w0
                        + expert(w1b, 1, w2b, 1) * w1)

    return _moe_body


@jax.jit
def kernel(x, c_states, router_W, router_b, W1, b1, W2, b2):
    B, T, D = x.shape
    E, _, H = W1.shape
    M = B * T
    BM = 1024
    NCH = 4

    idx2, wts2 = pl.pallas_call(
        _routing_body,
        out_shape=(
            jax.ShapeDtypeStruct((1, 2), jnp.int32),
            jax.ShapeDtypeStruct((1, 2), jnp.float32),
        ),
    )(c_states, router_W.T, router_b.reshape(1, E))
    idx = idx2.reshape(2)
    wts = wts2.reshape(2)

    x2 = x.reshape(M, D)
    out = pl.pallas_call(
        _make_moe_body(D, H, NCH),
        grid=(M // BM,),
        in_specs=[
            pl.BlockSpec(memory_space=pltpu.SMEM),
            pl.BlockSpec(memory_space=pltpu.SMEM),
            pl.BlockSpec((BM, D), lambda i: (i, 0)),
            pl.BlockSpec(memory_space=pltpu.MemorySpace.HBM),
            pl.BlockSpec(memory_space=pltpu.MemorySpace.HBM),
            pl.BlockSpec(memory_space=pltpu.MemorySpace.HBM),
            pl.BlockSpec(memory_space=pltpu.MemorySpace.HBM),
        ],
        out_specs=pl.BlockSpec((BM, D), lambda i: (i, 0)),
        out_shape=jax.ShapeDtypeStruct((M, D), jnp.float32),
        scratch_shapes=[
            pltpu.VMEM((D, H), jnp.bfloat16),
            pltpu.VMEM((D, H), jnp.bfloat16),
            pltpu.VMEM((H, D), jnp.bfloat16),
            pltpu.VMEM((H, D), jnp.bfloat16),
            pltpu.VMEM((2, 1, H), jnp.float32),
            pltpu.VMEM((2, 1, D), jnp.float32),
            pltpu.VMEM((2, D, H // NCH), jnp.float32),
            pltpu.SemaphoreType.DMA((2,)),
            pltpu.SemaphoreType.DMA,
        ],
        compiler_params=pltpu.CompilerParams(
            dimension_semantics=("arbitrary",),
        ),
    )(idx, wts, x2, W1, W2, b1.reshape(E, 1, H), b2.reshape(E, 1, D))
    return out.reshape(B, T, D)


# speedup vs baseline: 1.0763x; 1.0142x over previous
import jax
import jax.numpy as jnp
from jax.experimental import pallas as pl
from jax.experimental.pallas import tpu as pltpu

_INV_SQRT2 = 0.7071067811865476


def _make_moe_body(D, H, E, NCH):
    CH = H // NCH

    def _moe_body(c_ref, rwt_ref, rb_ref, x_ref, w1_hbm, w2_hbm, b1_hbm,
                  b2_hbm, out_ref, w1a, w1b, w2a, w2b, b1s, b2s, stage,
                  wsmem, sems, bsem):
        i = pl.program_id(0)

        @pl.when(i == 0)
        def _prologue():
            c_mean = jnp.mean(c_ref[...], axis=0, keepdims=True)
            logits = jnp.dot(
                c_mean.astype(jnp.bfloat16),
                rwt_ref[...].astype(jnp.bfloat16),
                preferred_element_type=jnp.float32,
            ) + rb_ref[...]
            lane = jax.lax.broadcasted_iota(jnp.int32, logits.shape, 1)
            m1 = jnp.max(logits)
            e0 = jnp.min(jnp.where(logits == m1, lane, E))
            masked = jnp.where(lane == e0, -jnp.inf, logits)
            m2 = jnp.max(masked)
            e1 = jnp.min(jnp.where(masked == m2, lane, E))
            t = jnp.exp(m2 - m1)
            wsmem[0] = 1.0 / (1.0 + t)
            wsmem[1] = t / (1.0 + t)
            bias_copies = [
                pltpu.make_async_copy(b1_hbm.at[e0], b1s.at[0], bsem),
                pltpu.make_async_copy(b1_hbm.at[e1], b1s.at[1], bsem),
                pltpu.make_async_copy(b2_hbm.at[e0], b2s.at[0], bsem),
                pltpu.make_async_copy(b2_hbm.at[e1], b2s.at[1], bsem),
            ]
            for c in bias_copies:
                c.start()
            chunks = []
            for e, w1d, w2d in ((e0, w1a, w2a), (e1, w1b, w2b)):
                for c in range(NCH):
                    sl = pl.ds(c * CH, CH)
                    chunks.append((w1_hbm.at[e, :, sl], w1d,
                                   (slice(None), sl)))
                for c in range(NCH):
                    sl = pl.ds(c * CH, CH)
                    chunks.append((w2_hbm.at[e, sl, :], w2d,
                                   (sl, slice(None))))
            copies = []
            for k, (src, _, _) in enumerate(chunks):
                copies.append(
                    pltpu.make_async_copy(src, stage.at[k % 2], sems.at[k % 2]))
            for k, (_, dst, dsl) in enumerate(chunks):
                copies[k].start()
                if k > 0:
                    copies[k - 1].wait()
                    _, pdst, pdsl = chunks[k - 1]
                    pdst[pdsl] = stage[(k - 1) % 2].astype(jnp.bfloat16)
            copies[-1].wait()
            _, ldst, ldsl = chunks[-1]
            ldst[ldsl] = stage[(len(chunks) - 1) % 2].astype(jnp.bfloat16)
            for c in bias_copies:
                c.wait()

        xv = x_ref[...].astype(jnp.bfloat16)

        def expert(w1_s, b1_i, w2_s, b2_i):
            h = jnp.dot(xv, w1_s[...], preferred_element_type=jnp.float32)
            h = h + b1s[b1_i]
            h = 0.5 * h * (1.0 + jax.lax.erf(h * _INV_SQRT2))
            return jnp.dot(h.astype(jnp.bfloat16), w2_s[...],
                           preferred_element_type=jnp.float32) + b2s[b2_i]

        w0 = wsmem[0]
        w1 = wsmem[1]
        out_ref[...] = (expert(w1a, 0, w2a, 0) * w0
                        + expert(w1b, 1, w2b, 1) * w1)

    return _moe_body


@jax.jit
def kernel(x, c_states, router_W, router_b, W1, b1, W2, b2):
    B, T, D = x.shape
    E, _, H = W1.shape
    N_CELLS, C = c_states.shape
    M = B * T
    BM = 1024
    NCH = 4

    x2 = x.reshape(M, D)
    out = pl.pallas_call(
        _make_moe_body(D, H, E, NCH),
        grid=(M // BM,),
        in_specs=[
            pl.BlockSpec((N_CELLS, C), lambda i: (0, 0)),
            pl.BlockSpec((C, E), lambda i: (0, 0)),
            pl.BlockSpec((1, E), lambda i: (0, 0)),
            pl.BlockSpec((BM, D), lambda i: (i, 0)),
            pl.BlockSpec(memory_space=pltpu.MemorySpace.HBM),
            pl.BlockSpec(memory_space=pltpu.MemorySpace.HBM),
            pl.BlockSpec(memory_space=pltpu.MemorySpace.HBM),
            pl.BlockSpec(memory_space=pltpu.MemorySpace.HBM),
        ],
        out_specs=pl.BlockSpec((BM, D), lambda i: (i, 0)),
        out_shape=jax.ShapeDtypeStruct((M, D), jnp.float32),
        scratch_shapes=[
            pltpu.VMEM((D, H), jnp.bfloat16),
            pltpu.VMEM((D, H), jnp.bfloat16),
            pltpu.VMEM((H, D), jnp.bfloat16),
            pltpu.VMEM((H, D), jnp.bfloat16),
            pltpu.VMEM((2, 1, H), jnp.float32),
            pltpu.VMEM((2, 1, D), jnp.float32),
            pltpu.VMEM((2, D, H // NCH), jnp.float32),
            pltpu.SMEM((2,), jnp.float32),
            pltpu.SemaphoreType.DMA((2,)),
            pltpu.SemaphoreType.DMA,
        ],
        compiler_params=pltpu.CompilerParams(
            dimension_semantics=("arbitrary",),
        ),
    )(c_states, router_W.T, router_b.reshape(1, E), x2, W1, W2,
      b1.reshape(E, 1, H), b2.reshape(E, 1, D))
    return out.reshape(B, T, D)
